# baseline (speedup 1.0000x reference)
import jax
import jax.numpy as jnp
from jax import lax
from jax.experimental import pallas as pl
from jax.experimental.pallas import tpu as pltpu
from jax.experimental.pallas import tpu_sc as plsc

_NUM_ENTITIES = 10000
_NUM_RELATIONS = 2000
_DIM = 128
_NUM_BASES = 4
_NUM_TRIPLETS = 16000

_NC = 2
_NS = 16
_NW = _NC * _NS
_LANES = 16
_CHUNK = _NUM_TRIPLETS // _NW
_GROUPS = -(-_CHUNK // _LANES)
_CHUNK_PAD = _GROUPS * _LANES
_SFLAT = _NUM_BASES * _DIM
_OUTROW = 640
_MATCH_CAP = 2 * _CHUNK_PAD + _LANES


def _sc_body(trip_hbm, u_hbm, att_hbm, ent_hbm, s_out,
             src_v, rel_v, dst_v, u_v, ind_v, glist_v,
             idx_v, rows_v, att_v, s_v, sem):
    wid = lax.axis_index("s") * _NC + lax.axis_index("c")
    lane = lax.iota(jnp.int32, _LANES)

    copies = [
        pltpu.async_copy(trip_hbm.at[0, wid], src_v, sem),
        pltpu.async_copy(trip_hbm.at[1, wid], rel_v, sem),
        pltpu.async_copy(trip_hbm.at[2, wid], dst_v, sem),
        pltpu.async_copy(u_hbm, u_v, sem),
    ]
    att_cp = pltpu.async_copy(att_hbm, att_v, sem)

    zeros16 = jnp.zeros((_LANES,), jnp.float32)

    def zbody(j, carry):
        s_v[pl.ds(pl.multiple_of(j * _LANES, _LANES), _LANES)] = zeros16
        return carry

    lax.fori_loop(0, _OUTROW // _LANES, zbody, 0)

    for c in copies:
        c.wait()
    u_vec = u_v[...]

    lane0 = lane == 0

    def group(g, tot):
        base = pl.multiple_of(g * _LANES, _LANES)
        s = src_v[pl.ds(base, _LANES)]
        r = rel_v[pl.ds(base, _LANES)]
        d = dst_v[pl.ds(base, _LANES)]
        m1 = d == u_vec
        m2 = s == u_vec
        pc = (plsc.all_reduce_population_count(m1)
              + plsc.all_reduce_population_count(m2))
        plsc.store_compressed(ind_v.at[pl.ds(g, _LANES)], pc, mask=lane0)
        return tot + pc

    tot = lax.fori_loop(0, _GROUPS, group,
                        jnp.zeros((_LANES,), jnp.int32))
    s_v[pl.ds(_SFLAT, _LANES)] = tot.astype(jnp.float32)

    c0 = ind_v[pl.ds(0, _LANES)]
    c1 = ind_v[pl.ds(_LANES, _LANES)]
    gm0 = c0 > 0
    gm1 = c1 > 0
    plsc.store_compressed(glist_v.at[pl.ds(0, _LANES)], lane, mask=gm0)
    n0 = jnp.sum(gm0.astype(jnp.int32))
    plsc.store_compressed(glist_v.at[pl.ds(n0, _LANES)], lane + _LANES,
                          mask=gm1)
    ng = n0 + jnp.sum(gm1.astype(jnp.int32))
    att_cp.wait()

    def batch(i, carry):
        gs = plsc.load_gather(glist_v, [jnp.zeros((_LANES,), jnp.int32) + i])
        g = jnp.sum(jnp.where(lane0, gs, 0))
        base = pl.multiple_of(g * _LANES, _LANES)
        s = src_v[pl.ds(base, _LANES)]
        r = rel_v[pl.ds(base, _LANES)]
        d = dst_v[pl.ds(base, _LANES)]
        m1 = d == u_vec
        m2 = s == u_vec

        def half(h, c2):
            mask = jnp.where(h == 0, m1, m2)
            gidx = jnp.where(mask, jnp.where(h == 0, s, d), 0)
            arow = jnp.where(mask, jnp.where(h == 0, r, r + _NUM_RELATIONS), 0)

            @pl.when(jnp.any(mask))
            def _():
                accumulate(gidx, arow, mask)

            return c2

        lax.fori_loop(0, 2, half, 0)
        return carry

    def accumulate(gidx, arow, mask):
        idx_v[...] = gidx
        pltpu.async_copy(ent_hbm.at[idx_v], rows_v, sem).wait()
        def bbody(b, carry_b):
            a_b = plsc.load_gather(
                att_v, [jnp.full((_LANES,), 0, jnp.int32) + b, arow])
            a_b = jnp.where(mask, a_b, 0.0)

            def mbody(m, accs):
                am = jnp.sum(jnp.where(lane == m, a_b, 0.0))
                return tuple(
                    accs[j] + am * rows_v[m, pl.ds(j * _LANES, _LANES)]
                    for j in range(_DIM // _LANES))

            accs = lax.fori_loop(0, _LANES, mbody,
                                 (zeros16,) * (_DIM // _LANES))
            for j in range(_DIM // _LANES):
                plsc.addupdate(
                    s_v.at[pl.ds(b * _DIM + j * _LANES, _LANES)], accs[j])
            return carry_b

        lax.fori_loop(0, _NUM_BASES, bbody, 0)

    lax.fori_loop(0, ng, batch, 0)

    pltpu.sync_copy(s_v, s_out.at[wid])


_sc_kernel = pl.kernel(
    _sc_body,
    out_type=jax.ShapeDtypeStruct((_NW, _OUTROW), jnp.float32),
    mesh=plsc.VectorSubcoreMesh(
        core_axis_name="c", subcore_axis_name="s",
        num_cores=_NC, num_subcores=_NS),
    scratch_types=[
        pltpu.VMEM((_CHUNK_PAD,), jnp.int32),
        pltpu.VMEM((_CHUNK_PAD,), jnp.int32),
        pltpu.VMEM((_CHUNK_PAD,), jnp.int32),
        pltpu.VMEM((_LANES,), jnp.int32),
        pltpu.VMEM((_GROUPS + _LANES,), jnp.int32),
        pltpu.VMEM((_GROUPS + _LANES,), jnp.int32),
        pltpu.VMEM((_LANES,), jnp.int32),
        pltpu.VMEM((_LANES, _DIM), jnp.float32),
        pltpu.VMEM((_NUM_BASES, 2 * _NUM_RELATIONS), jnp.float32),
        pltpu.VMEM((_OUTROW,), jnp.float32),
        pltpu.SemaphoreType.DMA,
    ],
    compiler_params=pltpu.CompilerParams(needs_layout_passes=False),
)


def _tc_body(s_ref, basis_ref, out_ref):
    s_sum = jnp.sum(s_ref[...], axis=0, keepdims=True)
    cnt = s_sum[0, _SFLAT]
    acc = jnp.zeros((1, _DIM), jnp.float32)
    for b in range(_NUM_BASES):
        sb = s_sum[:, b * _DIM:(b + 1) * _DIM]
        acc = acc + jnp.dot(sb, basis_ref[b],
                            preferred_element_type=jnp.float32)
    out_ref[...] = acc / jnp.maximum(cnt, 1.0)


@jax.jit
def kernel(triplets, unseen_entity, entity_embedding, basis, att):
    trip_t = triplets.astype(jnp.int32).T.reshape(3, _NW, _CHUNK)
    trip_t = jnp.pad(trip_t, ((0, 0), (0, 0), (0, _CHUNK_PAD - _CHUNK)),
                     constant_values=-1)
    u_splat = jnp.full((_LANES,), unseen_entity, dtype=jnp.int32)
    att_t = att.T
    s_all = _sc_kernel(trip_t, u_splat, att_t, entity_embedding)
    out = pl.pallas_call(
        _tc_body,
        out_shape=jax.ShapeDtypeStruct((1, _DIM), jnp.float32),
    )(s_all, basis)
    return out.reshape(_DIM)


# speedup vs baseline: 1.0566x; 1.0566x over previous
import jax
import jax.numpy as jnp
from jax import lax
from jax.experimental import pallas as pl
from jax.experimental.pallas import tpu as pltpu
from jax.experimental.pallas import tpu_sc as plsc

_NUM_ENTITIES = 10000
_NUM_RELATIONS = 2000
_DIM = 128
_NUM_BASES = 4
_NUM_TRIPLETS = 16000

_NC = 2
_NS = 16
_NW = _NC * _NS
_LANES = 16
_CHUNK = _NUM_TRIPLETS // _NW
_GROUPS = -(-_CHUNK // _LANES)
_CHUNK_PAD = _GROUPS * _LANES
_SFLAT = _NUM_BASES * _DIM
_OUTROW = 640
_MATCH_CAP = 2 * _CHUNK_PAD + _LANES


def _sc_body(trip_hbm, u_hbm, att_hbm, ent_hbm, s_out,
             src_v, rel_v, dst_v, u_v, gidx_v, arow_v,
             idx_v, rows_v, att_v, s_v, sem):
    wid = lax.axis_index("s") * _NC + lax.axis_index("c")
    lane = lax.iota(jnp.int32, _LANES)

    copies = [
        pltpu.async_copy(trip_hbm.at[0, wid], src_v, sem),
        pltpu.async_copy(trip_hbm.at[1, wid], rel_v, sem),
        pltpu.async_copy(trip_hbm.at[2, wid], dst_v, sem),
        pltpu.async_copy(u_hbm, u_v, sem),
    ]
    att_cp = pltpu.async_copy(att_hbm, att_v, sem)

    zeros16 = jnp.zeros((_LANES,), jnp.float32)

    def zbody(j, carry):
        s_v[pl.ds(pl.multiple_of(j * _LANES, _LANES), _LANES)] = zeros16
        return carry

    lax.fori_loop(0, _OUTROW // _LANES, zbody, 0)

    for c in copies:
        c.wait()
    u_vec = u_v[...]

    def group(g, n):
        base = pl.multiple_of(g * _LANES, _LANES)
        s = src_v[pl.ds(base, _LANES)]
        r = rel_v[pl.ds(base, _LANES)]
        d = dst_v[pl.ds(base, _LANES)]
        m1 = d == u_vec
        m2 = s == u_vec
        plsc.store_compressed(gidx_v.at[pl.ds(n, _LANES)], s, mask=m1)
        plsc.store_compressed(arow_v.at[pl.ds(n, _LANES)], r, mask=m1)
        n = n + jnp.sum(m1.astype(jnp.int32))
        plsc.store_compressed(gidx_v.at[pl.ds(n, _LANES)], d, mask=m2)
        plsc.store_compressed(arow_v.at[pl.ds(n, _LANES)],
                              r + _NUM_RELATIONS, mask=m2)
        n = n + jnp.sum(m2.astype(jnp.int32))
        return n

    n = lax.fori_loop(0, _GROUPS, group, jnp.int32(0), unroll=4)
    s_v[pl.ds(_SFLAT, _LANES)] = jnp.full((_LANES,), n).astype(jnp.float32)
    att_cp.wait()

    def batch(i, carry):
        base = pl.multiple_of(i * _LANES, _LANES)
        mask = (base + lane) < n
        gidx = jnp.where(mask, gidx_v[pl.ds(base, _LANES)], 0)
        arow = jnp.where(mask, arow_v[pl.ds(base, _LANES)], 0)
        idx_v[...] = gidx
        pltpu.async_copy(ent_hbm.at[idx_v], rows_v, sem).wait()
        def bbody(b, carry_b):
            a_b = plsc.load_gather(
                att_v, [jnp.full((_LANES,), 0, jnp.int32) + b, arow])
            a_b = jnp.where(mask, a_b, 0.0)

            def mbody(m, accs):
                am = jnp.sum(jnp.where(lane == m, a_b, 0.0))
                return tuple(
                    accs[j] + am * rows_v[m, pl.ds(j * _LANES, _LANES)]
                    for j in range(_DIM // _LANES))

            accs = lax.fori_loop(0, _LANES, mbody,
                                 (zeros16,) * (_DIM // _LANES))
            for j in range(_DIM // _LANES):
                plsc.addupdate(
                    s_v.at[pl.ds(b * _DIM + j * _LANES, _LANES)], accs[j])
            return carry_b

        lax.fori_loop(0, _NUM_BASES, bbody, 0)
        return carry

    lax.fori_loop(0, (n + _LANES - 1) // _LANES, batch, 0)

    pltpu.sync_copy(s_v, s_out.at[wid])


_sc_kernel = pl.kernel(
    _sc_body,
    out_type=jax.ShapeDtypeStruct((_NW, _OUTROW), jnp.float32),
    mesh=plsc.VectorSubcoreMesh(
        core_axis_name="c", subcore_axis_name="s",
        num_cores=_NC, num_subcores=_NS),
    scratch_types=[
        pltpu.VMEM((_CHUNK_PAD,), jnp.int32),
        pltpu.VMEM((_CHUNK_PAD,), jnp.int32),
        pltpu.VMEM((_CHUNK_PAD,), jnp.int32),
        pltpu.VMEM((_LANES,), jnp.int32),
        pltpu.VMEM((_MATCH_CAP,), jnp.int32),
        pltpu.VMEM((_MATCH_CAP,), jnp.int32),
        pltpu.VMEM((_LANES,), jnp.int32),
        pltpu.VMEM((_LANES, _DIM), jnp.float32),
        pltpu.VMEM((_NUM_BASES, 2 * _NUM_RELATIONS), jnp.float32),
        pltpu.VMEM((_OUTROW,), jnp.float32),
        pltpu.SemaphoreType.DMA,
    ],
    compiler_params=pltpu.CompilerParams(needs_layout_passes=False),
)


def _tc_body(s_ref, basis_ref, out_ref):
    s_sum = jnp.sum(s_ref[...], axis=0, keepdims=True)
    cnt = s_sum[0, _SFLAT]
    acc = jnp.zeros((1, _DIM), jnp.float32)
    for b in range(_NUM_BASES):
        sb = s_sum[:, b * _DIM:(b + 1) * _DIM]
        acc = acc + jnp.dot(sb, basis_ref[b],
                            preferred_element_type=jnp.float32)
    out_ref[...] = acc / jnp.maximum(cnt, 1.0)


@jax.jit
def kernel(triplets, unseen_entity, entity_embedding, basis, att):
    trip_t = triplets.astype(jnp.int32).T.reshape(3, _NW, _CHUNK)
    trip_t = jnp.pad(trip_t, ((0, 0), (0, 0), (0, _CHUNK_PAD - _CHUNK)),
                     constant_values=-1)
    u_splat = jnp.full((_LANES,), unseen_entity, dtype=jnp.int32)
    att_t = att.T
    s_all = _sc_kernel(trip_t, u_splat, att_t, entity_embedding)
    out = pl.pallas_call(
        _tc_body,
        out_shape=jax.ShapeDtypeStruct((1, _DIM), jnp.float32),
    )(s_all, basis)
    return out.reshape(_DIM)


# speedup vs baseline: 1.0813x; 1.0234x over previous
import jax
import jax.numpy as jnp
from jax import lax
from jax.experimental import pallas as pl
from jax.experimental.pallas import tpu as pltpu
from jax.experimental.pallas import tpu_sc as plsc

_NUM_ENTITIES = 10000
_NUM_RELATIONS = 2000
_DIM = 128
_NUM_BASES = 4
_NUM_TRIPLETS = 16000

_NC = 2
_NS = 16
_NW = _NC * _NS
_LANES = 16
_CHUNK = _NUM_TRIPLETS // _NW
_GROUPS = -(-_CHUNK // _LANES)
_CHUNK_PAD = _GROUPS * _LANES
_SFLAT = _NUM_BASES * _DIM
_OUTROW = 640
_MATCH_CAP = 2 * _CHUNK_PAD + _LANES


def _sc_body(trip_hbm, u_hbm, att_hbm, ent_hbm, s_out,
             src_v, rel_v, dst_v, u_v, gidx_v, arow_v,
             idx_v, rows_v, att_v, s_v, sem):
    wid = lax.axis_index("s") * _NC + lax.axis_index("c")
    lane = lax.iota(jnp.int32, _LANES)

    copies = [
        pltpu.async_copy(trip_hbm.at[0, wid], src_v, sem),
        pltpu.async_copy(trip_hbm.at[1, wid], rel_v, sem),
        pltpu.async_copy(trip_hbm.at[2, wid], dst_v, sem),
        pltpu.async_copy(u_hbm, u_v, sem),
    ]
    att_cp = pltpu.async_copy(att_hbm, att_v, sem)

    zeros16 = jnp.zeros((_LANES,), jnp.float32)

    def zbody(j, carry):
        s_v[pl.ds(pl.multiple_of(j * _LANES, _LANES), _LANES)] = zeros16
        return carry

    lax.fori_loop(0, _OUTROW // _LANES, zbody, 0)

    for c in copies:
        c.wait()
    u_vec = u_v[...]

    def group(g, n):
        base = pl.multiple_of(g * _LANES, _LANES)
        s = src_v[pl.ds(base, _LANES)]
        r = rel_v[pl.ds(base, _LANES)]
        d = dst_v[pl.ds(base, _LANES)]
        m1 = d == u_vec
        m2 = s == u_vec
        plsc.store_compressed(gidx_v.at[pl.ds(n, _LANES)], s, mask=m1)
        plsc.store_compressed(arow_v.at[pl.ds(n, _LANES)], r, mask=m1)
        n = n + jnp.sum(m1.astype(jnp.int32))
        plsc.store_compressed(gidx_v.at[pl.ds(n, _LANES)], d, mask=m2)
        plsc.store_compressed(arow_v.at[pl.ds(n, _LANES)],
                              r + _NUM_RELATIONS, mask=m2)
        n = n + jnp.sum(m2.astype(jnp.int32))
        return n

    n = lax.fori_loop(0, _GROUPS, group, jnp.int32(0), unroll=4)
    s_v[pl.ds(_SFLAT, _LANES)] = jnp.full((_LANES,), n).astype(jnp.float32)
    att_cp.wait()

    def batch(i, carry):
        base = pl.multiple_of(i * _LANES, _LANES)
        mask = (base + lane) < n
        gidx = jnp.where(mask, gidx_v[pl.ds(base, _LANES)], 0)
        arow = jnp.where(mask, arow_v[pl.ds(base, _LANES)], 0)
        idx_v[...] = gidx
        pltpu.async_copy(ent_hbm.at[idx_v], rows_v, sem).wait()
        def bbody(b, carry_b):
            a_b = plsc.load_gather(
                att_v, [jnp.full((_LANES,), 0, jnp.int32) + b, arow])
            a_b = jnp.where(mask, a_b, 0.0)

            def mbody(m, accs):
                am = jnp.sum(jnp.where(lane == m, a_b, 0.0))
                return tuple(
                    accs[j] + am * rows_v[m, pl.ds(j * _LANES, _LANES)]
                    for j in range(_DIM // _LANES))

            accs = lax.fori_loop(0, _LANES, mbody,
                                 (zeros16,) * (_DIM // _LANES))
            for j in range(_DIM // _LANES):
                plsc.addupdate(
                    s_v.at[pl.ds(b * _DIM + j * _LANES, _LANES)], accs[j])
            return carry_b

        lax.fori_loop(0, _NUM_BASES, bbody, 0)
        return carry

    lax.fori_loop(0, (n + _LANES - 1) // _LANES, batch, 0)

    pltpu.sync_copy(s_v, s_out.at[wid])


_sc_kernel = pl.kernel(
    _sc_body,
    out_type=jax.ShapeDtypeStruct((_NW, _OUTROW), jnp.float32),
    mesh=plsc.VectorSubcoreMesh(
        core_axis_name="c", subcore_axis_name="s",
        num_cores=_NC, num_subcores=_NS),
    scratch_types=[
        pltpu.VMEM((_CHUNK_PAD,), jnp.int32),
        pltpu.VMEM((_CHUNK_PAD,), jnp.int32),
        pltpu.VMEM((_CHUNK_PAD,), jnp.int32),
        pltpu.VMEM((_LANES,), jnp.int32),
        pltpu.VMEM((_MATCH_CAP,), jnp.int32),
        pltpu.VMEM((_MATCH_CAP,), jnp.int32),
        pltpu.VMEM((_LANES,), jnp.int32),
        pltpu.VMEM((_LANES, _DIM), jnp.float32),
        pltpu.VMEM((_NUM_BASES, 2 * _NUM_RELATIONS), jnp.float32),
        pltpu.VMEM((_OUTROW,), jnp.float32),
        pltpu.SemaphoreType.DMA,
    ],
    compiler_params=pltpu.CompilerParams(needs_layout_passes=False),
)


def _tc_body(s_ref, basis_ref, out_ref):
    s_sum = jnp.sum(s_ref[...], axis=0, keepdims=True)
    cnt = s_sum[0, _SFLAT]
    acc = jnp.zeros((1, _DIM), jnp.float32)
    for b in range(_NUM_BASES):
        sb = s_sum[:, b * _DIM:(b + 1) * _DIM]
        acc = acc + jnp.dot(sb, basis_ref[b],
                            preferred_element_type=jnp.float32)
    out_ref[...] = acc / jnp.maximum(cnt, 1.0)


@jax.jit
def kernel(triplets, unseen_entity, entity_embedding, basis, att):
    trip_t = jnp.pad(triplets.astype(jnp.int32).T,
                     ((0, 0), (0, _NW * _CHUNK_PAD - _NUM_TRIPLETS)),
                     constant_values=-1).reshape(3, _NW, _CHUNK_PAD)
    u_splat = jnp.full((_LANES,), unseen_entity, dtype=jnp.int32)
    att_t = att.T
    s_all = _sc_kernel(trip_t, u_splat, att_t, entity_embedding)
    out = pl.pallas_call(
        _tc_body,
        out_shape=jax.ShapeDtypeStruct((1, _DIM), jnp.float32),
    )(s_all, basis)
    return out.reshape(_DIM)


# speedup vs baseline: 1.2169x; 1.1255x over previous
import jax
import jax.numpy as jnp
from jax import lax
from jax.experimental import pallas as pl
from jax.experimental.pallas import tpu as pltpu
from jax.experimental.pallas import tpu_sc as plsc

_NUM_ENTITIES = 10000
_NUM_RELATIONS = 2000
_DIM = 128
_NUM_BASES = 4
_NUM_TRIPLETS = 16000

_NC = 2
_NS = 16
_NW = _NC * _NS
_LANES = 16
_CHUNK = _NUM_TRIPLETS // _NW
_GROUPS = -(-_CHUNK // _LANES)
_CHUNK_PAD = _GROUPS * _LANES
_SFLAT = _NUM_BASES * _DIM
_OUTROW = 640
_MATCH_CAP = 2 * _CHUNK_PAD + _LANES


def _sc_body(trip_hbm, u_hbm, att_hbm, ent_hbm, s_out,
             src_v, rel_v, dst_v, u_v, gidx_v, arow_v,
             idx_v, rows_v, att_v, s_v, sem):
    wid = lax.axis_index("s") * _NC + lax.axis_index("c")
    lane = lax.iota(jnp.int32, _LANES)

    copies = [
        pltpu.async_copy(trip_hbm.at[0, wid], src_v, sem),
        pltpu.async_copy(trip_hbm.at[1, wid], rel_v, sem),
        pltpu.async_copy(trip_hbm.at[2, wid], dst_v, sem),
        pltpu.async_copy(u_hbm, u_v, sem),
    ]
    att_cp = pltpu.async_copy(att_hbm, att_v, sem)

    zeros16 = jnp.zeros((_LANES,), jnp.float32)

    def zbody(j, carry):
        s_v[pl.ds(pl.multiple_of(j * _LANES, _LANES), _LANES)] = zeros16
        return carry

    lax.fori_loop(0, _OUTROW // _LANES, zbody, 0)

    for c in copies:
        c.wait()
    u_vec = plsc.load_gather(u_v, [jnp.zeros((_LANES,), jnp.int32)])

    def group(g, n):
        base = pl.multiple_of(g * _LANES, _LANES)
        s = src_v[pl.ds(base, _LANES)]
        r = rel_v[pl.ds(base, _LANES)]
        d = dst_v[pl.ds(base, _LANES)]
        m1 = d == u_vec
        m2 = s == u_vec
        plsc.store_compressed(gidx_v.at[pl.ds(n, _LANES)], s, mask=m1)
        plsc.store_compressed(arow_v.at[pl.ds(n, _LANES)], r, mask=m1)
        n = n + jnp.sum(m1.astype(jnp.int32))
        plsc.store_compressed(gidx_v.at[pl.ds(n, _LANES)], d, mask=m2)
        plsc.store_compressed(arow_v.at[pl.ds(n, _LANES)],
                              r + _NUM_RELATIONS, mask=m2)
        n = n + jnp.sum(m2.astype(jnp.int32))
        return n

    n = lax.fori_loop(0, _GROUPS, group, jnp.int32(0), unroll=4)
    s_v[pl.ds(_SFLAT, _LANES)] = jnp.full((_LANES,), n).astype(jnp.float32)
    att_cp.wait()

    def batch(i, carry):
        base = pl.multiple_of(i * _LANES, _LANES)
        mask = (base + lane) < n
        gidx = jnp.where(mask, gidx_v[pl.ds(base, _LANES)], 0)
        arow = jnp.where(mask, arow_v[pl.ds(base, _LANES)], 0)
        idx_v[...] = gidx
        pltpu.async_copy(ent_hbm.at[idx_v], rows_v, sem).wait()
        def bbody(b, carry_b):
            a_b = plsc.load_gather(
                att_v, [jnp.full((_LANES,), 0, jnp.int32) + b, arow])
            a_b = jnp.where(mask, a_b, 0.0)

            def mbody(m, accs):
                am = jnp.sum(jnp.where(lane == m, a_b, 0.0))
                return tuple(
                    accs[j] + am * rows_v[m, pl.ds(j * _LANES, _LANES)]
                    for j in range(_DIM // _LANES))

            accs = lax.fori_loop(0, _LANES, mbody,
                                 (zeros16,) * (_DIM // _LANES))
            for j in range(_DIM // _LANES):
                plsc.addupdate(
                    s_v.at[pl.ds(b * _DIM + j * _LANES, _LANES)], accs[j])
            return carry_b

        lax.fori_loop(0, _NUM_BASES, bbody, 0)
        return carry

    lax.fori_loop(0, (n + _LANES - 1) // _LANES, batch, 0)

    pltpu.sync_copy(s_v, s_out.at[wid])


_sc_kernel = pl.kernel(
    _sc_body,
    out_type=jax.ShapeDtypeStruct((_NW, _OUTROW), jnp.float32),
    mesh=plsc.VectorSubcoreMesh(
        core_axis_name="c", subcore_axis_name="s",
        num_cores=_NC, num_subcores=_NS),
    scratch_types=[
        pltpu.VMEM((_CHUNK_PAD,), jnp.int32),
        pltpu.VMEM((_CHUNK_PAD,), jnp.int32),
        pltpu.VMEM((_CHUNK_PAD,), jnp.int32),
        pltpu.VMEM((1,), jnp.int32),
        pltpu.VMEM((_MATCH_CAP,), jnp.int32),
        pltpu.VMEM((_MATCH_CAP,), jnp.int32),
        pltpu.VMEM((_LANES,), jnp.int32),
        pltpu.VMEM((_LANES, _DIM), jnp.float32),
        pltpu.VMEM((_NUM_BASES, 2 * _NUM_RELATIONS), jnp.float32),
        pltpu.VMEM((_OUTROW,), jnp.float32),
        pltpu.SemaphoreType.DMA,
    ],
    compiler_params=pltpu.CompilerParams(needs_layout_passes=False),
)


def _tc_body(s_ref, basis_ref, out_ref):
    s_sum = jnp.sum(s_ref[...], axis=0, keepdims=True)
    cnt = s_sum[0, _SFLAT]
    acc = jnp.zeros((1, _DIM), jnp.float32)
    for b in range(_NUM_BASES):
        sb = s_sum[:, b * _DIM:(b + 1) * _DIM]
        acc = acc + jnp.dot(sb, basis_ref[b],
                            preferred_element_type=jnp.float32)
    out_ref[...] = acc / jnp.maximum(cnt, 1.0)


@jax.jit
def kernel(triplets, unseen_entity, entity_embedding, basis, att):
    trip_t = jnp.pad(triplets.astype(jnp.int32).T,
                     ((0, 0), (0, _NW * _CHUNK_PAD - _NUM_TRIPLETS)),
                     constant_values=-1).reshape(3, _NW, _CHUNK_PAD)
    u_splat = jnp.asarray(unseen_entity, dtype=jnp.int32).reshape(1)
    att_t = att.T
    s_all = _sc_kernel(trip_t, u_splat, att_t, entity_embedding)
    out = pl.pallas_call(
        _tc_body,
        out_shape=jax.ShapeDtypeStruct((1, _DIM), jnp.float32),
    )(s_all, basis)
    return out.reshape(_DIM)
